# Initial kernel scaffold; baseline (speedup 1.0000x reference)
import jax
import jax.numpy as jnp
from jax.experimental import pallas as pl


def kernel(x, edge_index, batch, conv0_W1, conv0_b1, conv0_gamma, conv0_beta, conv0_W2, conv0_b2, conv1_W1, conv1_b1, conv1_gamma, conv1_beta, conv1_W2, conv1_b2, conv2_W1, conv2_b1, conv2_gamma, conv2_beta, conv2_W2, conv2_b2, mlp_W1, mlp_b1, mlp_W2, mlp_b2):
    raise NotImplementedError("write your pallas kernel here")


# speedup vs baseline: 6.0390x; 6.0390x over previous
import functools

import jax
import jax.numpy as jnp
from jax import lax
from jax.experimental import pallas as pl
from jax.experimental.pallas import tpu as pltpu
from jax.experimental.pallas import tpu_sc as plsc

_N = 10000
_E = 320000
_D = 128
_G = 64

_NC = 2
_NS = 16
_NW = _NC * _NS
_EPT = _E // _NW
_CHUNK = 80
_NCHUNK = _EPT // _CHUNK

_BLK = 400
_NBLK = _N // _BLK


@functools.cache
def _get_sc_aggregate():
    mesh = plsc.VectorSubcoreMesh(
        core_axis_name="c", subcore_axis_name="s",
        num_cores=_NC, num_subcores=_NS)

    @functools.partial(
        pl.kernel,
        out_type=jax.ShapeDtypeStruct((2, _N, _D), jnp.float32),
        mesh=mesh,
        scratch_types=[
            pltpu.VMEM((_NCHUNK, _CHUNK), jnp.int32),
            pltpu.VMEM((_NCHUNK, _CHUNK), jnp.int32),
            pltpu.VMEM((_CHUNK, _D), jnp.float32),
            pltpu.VMEM_SHARED((_N, _D), jnp.float32),
            pltpu.SemaphoreType.DMA,
        ],
    )
    def sc_aggregate(h_hbm, zeros_hbm, src_hbm, dst_hbm, out_hbm,
                     src_v, dst_v, rows_v, agg_sh, gsem):
        c = lax.axis_index("c")
        s = lax.axis_index("s")
        wid = s * _NC + c

        pltpu.sync_copy(src_hbm.at[wid], src_v)
        pltpu.sync_copy(dst_hbm.at[wid], dst_v)

        @pl.when(s == 0)
        def _():
            @pl.when(c == 0)
            def _():
                pltpu.sync_copy(h_hbm, agg_sh)

            @pl.when(c == 1)
            def _():
                pltpu.sync_copy(zeros_hbm, agg_sh)

        plsc.subcore_barrier()

        def body(j, carry):
            pltpu.async_copy(h_hbm.at[src_v.at[j]], rows_v, gsem).wait()
            pltpu.sync_copy(rows_v, agg_sh.at[dst_v.at[j]], add=True)
            return carry

        lax.fori_loop(0, _NCHUNK, body, 0)

        plsc.subcore_barrier()

        @pl.when(s == 0)
        def _():
            pltpu.sync_copy(agg_sh, out_hbm.at[c])

    return sc_aggregate


def _tc_mlp_body(p_ref, W1_ref, b1_ref, gamma_ref, beta_ref, W2_ref, b2_ref,
                 out_ref, t_sc, sum_sc, sq_sc, ss_sc):
    ph = pl.program_id(0)
    i = pl.program_id(1)

    @pl.when(ph == 0)
    def _():
        h = p_ref[0] + p_ref[1]
        t = jnp.dot(h, W1_ref[...], preferred_element_type=jnp.float32)
        t = t + b1_ref[...]
        t_sc[pl.ds(i * _BLK, _BLK), :] = t

        @pl.when(i == 0)
        def _():
            sum_sc[...] = jnp.zeros_like(sum_sc)
            sq_sc[...] = jnp.zeros_like(sq_sc)

        sum_sc[...] += jnp.sum(t, axis=0, keepdims=True)
        sq_sc[...] += jnp.sum(t * t, axis=0, keepdims=True)

        @pl.when(i == _NBLK - 1)
        def _():
            mean = sum_sc[...] / _N
            var = sq_sc[...] / _N - mean * mean
            scale = gamma_ref[...] * lax.rsqrt(var + 1e-5)
            ss_sc[0:1, :] = scale
            ss_sc[1:2, :] = beta_ref[...] - mean * scale

    @pl.when(ph == 1)
    def _():
        t = t_sc[pl.ds(i * _BLK, _BLK), :]
        u = jnp.maximum(t * ss_sc[0:1, :] + ss_sc[1:2, :], 0.0)
        y = jnp.dot(u, W2_ref[...], preferred_element_type=jnp.float32)
        out_ref[...] = jnp.maximum(y + b2_ref[...], 0.0)


_vec_spec = pl.BlockSpec((1, _D), lambda ph, i: (0, 0))
_mat_spec = pl.BlockSpec((_D, _D), lambda ph, i: (0, 0))
_p_spec = pl.BlockSpec((2, _BLK, _D), lambda ph, i: (0, i, 0))

_tc_mlp = pl.pallas_call(
    _tc_mlp_body,
    grid=(2, _NBLK),
    in_specs=[_p_spec, _mat_spec, _vec_spec, _vec_spec, _vec_spec, _mat_spec,
              _vec_spec],
    out_specs=pl.BlockSpec((_BLK, _D), lambda ph, i: (i, 0)),
    out_shape=jax.ShapeDtypeStruct((_N, _D), jnp.float32),
    scratch_shapes=[
        pltpu.VMEM((_N, _D), jnp.float32),
        pltpu.VMEM((1, _D), jnp.float32),
        pltpu.VMEM((1, _D), jnp.float32),
        pltpu.VMEM((2, _D), jnp.float32),
    ],
    compiler_params=pltpu.CompilerParams(
        dimension_semantics=("arbitrary", "arbitrary")),
)


def _tc_final_body(p_ref, W1_ref, b1_ref, gamma_ref, beta_ref, W2_ref, b2_ref,
                   bm_ref, mW1_ref, mb1_ref, mW2_ref, mb2_ref,
                   out_ref, t_sc, sum_sc, sq_sc, ss_sc, pool_sc):
    ph = pl.program_id(0)
    i = pl.program_id(1)

    @pl.when(ph == 0)
    def _():
        h = p_ref[0] + p_ref[1]
        t = jnp.dot(h, W1_ref[...], preferred_element_type=jnp.float32)
        t = t + b1_ref[...]
        t_sc[pl.ds(i * _BLK, _BLK), :] = t

        @pl.when(i == 0)
        def _():
            sum_sc[...] = jnp.zeros_like(sum_sc)
            sq_sc[...] = jnp.zeros_like(sq_sc)

        sum_sc[...] += jnp.sum(t, axis=0, keepdims=True)
        sq_sc[...] += jnp.sum(t * t, axis=0, keepdims=True)

        @pl.when(i == _NBLK - 1)
        def _():
            mean = sum_sc[...] / _N
            var = sq_sc[...] / _N - mean * mean
            scale = gamma_ref[...] * lax.rsqrt(var + 1e-5)
            ss_sc[0:1, :] = scale
            ss_sc[1:2, :] = beta_ref[...] - mean * scale

    @pl.when(ph == 1)
    def _():
        t = t_sc[pl.ds(i * _BLK, _BLK), :]
        u = jnp.maximum(t * ss_sc[0:1, :] + ss_sc[1:2, :], 0.0)
        y = jnp.dot(u, W2_ref[...], preferred_element_type=jnp.float32)
        y = jnp.maximum(y + b2_ref[...], 0.0)

        oh = (bm_ref[0] == lax.broadcasted_iota(jnp.int32, (_G, _BLK), 0))
        contrib = jnp.dot(oh.astype(jnp.float32), y,
                          preferred_element_type=jnp.float32)

        @pl.when(i == 0)
        def _():
            pool_sc[...] = contrib

        @pl.when(i > 0)
        def _():
            pool_sc[...] += contrib

        @pl.when(i == _NBLK - 1)
        def _():
            h2 = jnp.dot(pool_sc[...], mW1_ref[...],
                         preferred_element_type=jnp.float32)
            h2 = jnp.maximum(h2 + mb1_ref[...], 0.0)
            o = jnp.dot(h2, mW2_ref[...], preferred_element_type=jnp.float32)
            out_ref[...] = o + mb2_ref[...]


_tc_final = pl.pallas_call(
    _tc_final_body,
    grid=(2, _NBLK),
    in_specs=[_p_spec, _mat_spec, _vec_spec, _vec_spec, _vec_spec, _mat_spec,
              _vec_spec,
              pl.BlockSpec((1, 1, _BLK), lambda ph, i: (i, 0, 0)),
              _mat_spec, _vec_spec, _mat_spec, _vec_spec],
    out_specs=pl.BlockSpec((_G, _D), lambda ph, i: (0, 0)),
    out_shape=jax.ShapeDtypeStruct((_G, _D), jnp.float32),
    scratch_shapes=[
        pltpu.VMEM((_N, _D), jnp.float32),
        pltpu.VMEM((1, _D), jnp.float32),
        pltpu.VMEM((1, _D), jnp.float32),
        pltpu.VMEM((2, _D), jnp.float32),
        pltpu.VMEM((_G, _D), jnp.float32),
    ],
    compiler_params=pltpu.CompilerParams(
        dimension_semantics=("arbitrary", "arbitrary")),
)


def kernel(x, edge_index, batch,
           conv0_W1, conv0_b1, conv0_gamma, conv0_beta, conv0_W2, conv0_b2,
           conv1_W1, conv1_b1, conv1_gamma, conv1_beta, conv1_W2, conv1_b2,
           conv2_W1, conv2_b1, conv2_gamma, conv2_beta, conv2_W2, conv2_b2,
           mlp_W1, mlp_b1, mlp_W2, mlp_b2):
    src = edge_index[0].reshape(_NW, _NCHUNK, _CHUNK)
    dst = edge_index[1].reshape(_NW, _NCHUNK, _CHUNK)
    zeros = jnp.zeros((_N, _D), jnp.float32)
    bm = batch.reshape(_NBLK, 1, _BLK)
    r1 = lambda v: v.reshape(1, _D)

    sc_aggregate = _get_sc_aggregate()
    h = x
    for (W1, b1, g, be, W2, b2) in (
        (conv0_W1, conv0_b1, conv0_gamma, conv0_beta, conv0_W2, conv0_b2),
        (conv1_W1, conv1_b1, conv1_gamma, conv1_beta, conv1_W2, conv1_b2),
    ):
        parts = sc_aggregate(h, zeros, src, dst)
        h = _tc_mlp(parts, W1, r1(b1), r1(g), r1(be), W2, r1(b2))

    parts = sc_aggregate(h, zeros, src, dst)
    out = _tc_final(parts, conv2_W1, r1(conv2_b1), r1(conv2_gamma),
                    r1(conv2_beta), conv2_W2, r1(conv2_b2),
                    bm, mlp_W1, r1(mlp_b1), mlp_W2, r1(mlp_b2))
    return out


# speedup vs baseline: 8.0669x; 1.3358x over previous
import functools

import jax
import jax.numpy as jnp
from jax import lax
from jax.experimental import pallas as pl
from jax.experimental.pallas import tpu as pltpu
from jax.experimental.pallas import tpu_sc as plsc

_N = 10000
_E = 320000
_D = 128
_G = 64

_NC = 2
_NS = 16
_NW = _NC * _NS
_EPT = _E // _NW
_CHUNK = 80
_NCHUNK = _EPT // _CHUNK
_W = 25
_NWIN = _NCHUNK // _W

_BLK = 1000
_NBLK = _N // _BLK
_RPT = 624
_RREM = _N - _NS * _RPT


@functools.cache
def _get_sc_aggregate():
    mesh = plsc.VectorSubcoreMesh(
        core_axis_name="c", subcore_axis_name="s",
        num_cores=_NC, num_subcores=_NS)

    @functools.partial(
        pl.kernel,
        out_type=jax.ShapeDtypeStruct((2, _N, _D), jnp.float32),
        mesh=mesh,
        scratch_types=[
            pltpu.VMEM((2, _W, 2, _CHUNK), jnp.int32),
            pltpu.VMEM((2, _CHUNK, _D), jnp.float32),
            pltpu.VMEM_SHARED((_N, _D), jnp.float32),
            pltpu.SemaphoreType.DMA,
            pltpu.SemaphoreType.DMA,
            pltpu.SemaphoreType.DMA,
        ],
    )
    def sc_aggregate(h_hbm, zeros_hbm, eidx_hbm, out_hbm,
                     widx, rows_v, agg_sh, isem, gsem0, gsem1):
        c = lax.axis_index("c")
        s = lax.axis_index("s")
        wid = s * _NC + c

        @pl.when(s == 0)
        def _():
            @pl.when(c == 0)
            def _():
                pltpu.sync_copy(h_hbm, agg_sh)

            @pl.when(c == 1)
            def _():
                pltpu.sync_copy(zeros_hbm, agg_sh)

        plsc.subcore_barrier()

        def gather(ib, j, b, sem):
            return pltpu.async_copy(h_hbm.at[widx.at[ib, j, 0]],
                                    rows_v.at[b], sem)

        def gwait(b, sem):
            pltpu.make_async_copy(h_hbm.at[widx.at[0, 0, 0]],
                                  rows_v.at[b], sem).wait()

        def scatter(ib, j, b):
            pltpu.sync_copy(rows_v.at[b], agg_sh.at[widx.at[ib, j, 1]],
                            add=True)

        pltpu.sync_copy(eidx_hbm.at[wid, 0], widx.at[0])
        for w in range(_NWIN):
            ib = w % 2
            if w > 0:
                pltpu.make_async_copy(eidx_hbm.at[wid, w], widx.at[ib],
                                      isem).wait()
            if w + 1 < _NWIN:
                pltpu.async_copy(eidx_hbm.at[wid, w + 1], widx.at[1 - ib],
                                 isem)

            gather(ib, 0, 0, gsem0)

            def body(k, carry):
                j0 = 2 * k
                j1 = j0 + 1
                gwait(0, gsem0)
                gather(ib, j1, 1, gsem1)
                scatter(ib, j0, 0)
                gwait(1, gsem1)
                gather(ib, j1 + 1, 0, gsem0)
                scatter(ib, j1, 1)
                return carry

            lax.fori_loop(0, (_W - 1) // 2, body, 0)
            gwait(0, gsem0)
            scatter(ib, _W - 1, 0)

        plsc.subcore_barrier()

        @pl.when(s == 0)
        def _():
            pltpu.sync_copy(agg_sh, out_hbm.at[c])

    return sc_aggregate


def _tc_mlp_body(p_ref, W1_ref, b1_ref, gamma_ref, beta_ref, W2_ref, b2_ref,
                 out_ref, t_sc, sum_sc, sq_sc, ss_sc):
    ph = pl.program_id(0)
    i = pl.program_id(1)

    @pl.when(ph == 0)
    def _():
        h = p_ref[0] + p_ref[1]
        t = jnp.dot(h, W1_ref[...], preferred_element_type=jnp.float32)
        t = t + b1_ref[...]
        t_sc[pl.ds(i * _BLK, _BLK), :] = t

        @pl.when(i == 0)
        def _():
            sum_sc[...] = jnp.zeros_like(sum_sc)
            sq_sc[...] = jnp.zeros_like(sq_sc)

        sum_sc[...] += jnp.sum(t, axis=0, keepdims=True)
        sq_sc[...] += jnp.sum(t * t, axis=0, keepdims=True)

        @pl.when(i == _NBLK - 1)
        def _():
            mean = sum_sc[...] / _N
            var = sq_sc[...] / _N - mean * mean
            scale = gamma_ref[...] * lax.rsqrt(var + 1e-5)
            ss_sc[0:1, :] = scale
            ss_sc[1:2, :] = beta_ref[...] - mean * scale

    @pl.when(ph == 1)
    def _():
        t = t_sc[pl.ds(i * _BLK, _BLK), :]
        u = jnp.maximum(t * ss_sc[0:1, :] + ss_sc[1:2, :], 0.0)
        y = jnp.dot(u, W2_ref[...], preferred_element_type=jnp.float32)
        out_ref[...] = jnp.maximum(y + b2_ref[...], 0.0)


_vec_spec = pl.BlockSpec((1, _D), lambda ph, i: (0, 0))
_mat_spec = pl.BlockSpec((_D, _D), lambda ph, i: (0, 0))
_p_spec = pl.BlockSpec((2, _BLK, _D), lambda ph, i: (0, i * (1 - ph), 0))

_tc_mlp = pl.pallas_call(
    _tc_mlp_body,
    grid=(2, _NBLK),
    in_specs=[_p_spec, _mat_spec, _vec_spec, _vec_spec, _vec_spec, _mat_spec,
              _vec_spec],
    out_specs=pl.BlockSpec((_BLK, _D), lambda ph, i: (i, 0)),
    out_shape=jax.ShapeDtypeStruct((_N, _D), jnp.float32),
    scratch_shapes=[
        pltpu.VMEM((_N, _D), jnp.float32),
        pltpu.VMEM((1, _D), jnp.float32),
        pltpu.VMEM((1, _D), jnp.float32),
        pltpu.VMEM((2, _D), jnp.float32),
    ],
    compiler_params=pltpu.CompilerParams(
        dimension_semantics=("arbitrary", "arbitrary")),
)


def _tc_final_body(p_ref, W1_ref, b1_ref, gamma_ref, beta_ref, W2_ref, b2_ref,
                   bm_ref, mW1_ref, mb1_ref, mW2_ref, mb2_ref,
                   out_ref, t_sc, sum_sc, sq_sc, ss_sc, pool_sc):
    ph = pl.program_id(0)
    i = pl.program_id(1)

    @pl.when(ph == 0)
    def _():
        h = p_ref[0] + p_ref[1]
        t = jnp.dot(h, W1_ref[...], preferred_element_type=jnp.float32)
        t = t + b1_ref[...]
        t_sc[pl.ds(i * _BLK, _BLK), :] = t

        @pl.when(i == 0)
        def _():
            sum_sc[...] = jnp.zeros_like(sum_sc)
            sq_sc[...] = jnp.zeros_like(sq_sc)

        sum_sc[...] += jnp.sum(t, axis=0, keepdims=True)
        sq_sc[...] += jnp.sum(t * t, axis=0, keepdims=True)

        @pl.when(i == _NBLK - 1)
        def _():
            mean = sum_sc[...] / _N
            var = sq_sc[...] / _N - mean * mean
            scale = gamma_ref[...] * lax.rsqrt(var + 1e-5)
            ss_sc[0:1, :] = scale
            ss_sc[1:2, :] = beta_ref[...] - mean * scale

    @pl.when(ph == 1)
    def _():
        t = t_sc[pl.ds(i * _BLK, _BLK), :]
        u = jnp.maximum(t * ss_sc[0:1, :] + ss_sc[1:2, :], 0.0)
        y = jnp.dot(u, W2_ref[...], preferred_element_type=jnp.float32)
        y = jnp.maximum(y + b2_ref[...], 0.0)

        oh = (bm_ref[0] == lax.broadcasted_iota(jnp.int32, (_G, _BLK), 0))
        contrib = jnp.dot(oh.astype(jnp.float32), y,
                          preferred_element_type=jnp.float32)

        @pl.when(i == 0)
        def _():
            pool_sc[...] = contrib

        @pl.when(i > 0)
        def _():
            pool_sc[...] += contrib

        @pl.when(i == _NBLK - 1)
        def _():
            h2 = jnp.dot(pool_sc[...], mW1_ref[...],
                         preferred_element_type=jnp.float32)
            h2 = jnp.maximum(h2 + mb1_ref[...], 0.0)
            o = jnp.dot(h2, mW2_ref[...], preferred_element_type=jnp.float32)
            out_ref[...] = o + mb2_ref[...]


_tc_final = pl.pallas_call(
    _tc_final_body,
    grid=(2, _NBLK),
    in_specs=[_p_spec, _mat_spec, _vec_spec, _vec_spec, _vec_spec, _mat_spec,
              _vec_spec,
              pl.BlockSpec((1, 1, _BLK), lambda ph, i: (i, 0, 0)),
              _mat_spec, _vec_spec, _mat_spec, _vec_spec],
    out_specs=pl.BlockSpec((_G, _D), lambda ph, i: (0, 0)),
    out_shape=jax.ShapeDtypeStruct((_G, _D), jnp.float32),
    scratch_shapes=[
        pltpu.VMEM((_N, _D), jnp.float32),
        pltpu.VMEM((1, _D), jnp.float32),
        pltpu.VMEM((1, _D), jnp.float32),
        pltpu.VMEM((2, _D), jnp.float32),
        pltpu.VMEM((_G, _D), jnp.float32),
    ],
    compiler_params=pltpu.CompilerParams(
        dimension_semantics=("arbitrary", "arbitrary")),
)


def kernel(x, edge_index, batch,
           conv0_W1, conv0_b1, conv0_gamma, conv0_beta, conv0_W2, conv0_b2,
           conv1_W1, conv1_b1, conv1_gamma, conv1_beta, conv1_W2, conv1_b2,
           conv2_W1, conv2_b1, conv2_gamma, conv2_beta, conv2_W2, conv2_b2,
           mlp_W1, mlp_b1, mlp_W2, mlp_b2):
    src = edge_index[0].reshape(_NW, _NCHUNK, _CHUNK)
    dst = edge_index[1].reshape(_NW, _NCHUNK, _CHUNK)
    eidx = jnp.stack([src, dst], axis=2).reshape(_NW, _NWIN, _W, 2, _CHUNK)
    zeros = jnp.zeros((_N, _D), jnp.float32)
    bm = batch.reshape(_NBLK, 1, _BLK)
    r1 = lambda v: v.reshape(1, _D)

    sc_aggregate = _get_sc_aggregate()
    h = x
    for (W1, b1, g, be, W2, b2) in (
        (conv0_W1, conv0_b1, conv0_gamma, conv0_beta, conv0_W2, conv0_b2),
        (conv1_W1, conv1_b1, conv1_gamma, conv1_beta, conv1_W2, conv1_b2),
    ):
        parts = sc_aggregate(h, zeros, eidx)
        h = _tc_mlp(parts, W1, r1(b1), r1(g), r1(be), W2, r1(b2))

    parts = sc_aggregate(h, zeros, eidx)
    out = _tc_final(parts, conv2_W1, r1(conv2_b1), r1(conv2_gamma),
                    r1(conv2_beta), conv2_W2, r1(conv2_b2),
                    bm, mlp_W1, r1(mlp_b1), mlp_W2, r1(mlp_b2))
    return out


# speedup vs baseline: 12.1850x; 1.5105x over previous
import functools

import jax
import jax.numpy as jnp
from jax import lax
from jax.experimental import pallas as pl
from jax.experimental.pallas import tpu as pltpu
from jax.experimental.pallas import tpu_sc as plsc

_N = 10000
_E = 320000
_D = 128
_G = 64

_NC = 2
_NS = 16
_NW = _NC * _NS
_EPT = _E // _NW
_CHUNK = 80
_NCHUNK = _EPT // _CHUNK
_W = 25
_NWIN = _NCHUNK // _W

_BLK = 1000
_NBLK = _N // _BLK
_RPT = 624
_RREM = _N - _NS * _RPT


@functools.cache
def _get_sc_aggregate():
    mesh = plsc.VectorSubcoreMesh(
        core_axis_name="c", subcore_axis_name="s",
        num_cores=_NC, num_subcores=_NS)

    @functools.partial(
        pl.kernel,
        out_type=jax.ShapeDtypeStruct((2, _N, _D), jnp.float32),
        mesh=mesh,
        scratch_types=[
            pltpu.VMEM((_EPT,), jnp.int32),
            pltpu.VMEM((2, _W, _CHUNK), jnp.int32),
            pltpu.VMEM((3, _CHUNK, _D), jnp.float32),
            pltpu.VMEM_SHARED((_N, _D), jnp.float32),
            pltpu.SemaphoreType.DMA,
            pltpu.SemaphoreType.DMA,
            pltpu.SemaphoreType.DMA,
            pltpu.SemaphoreType.DMA,
        ],
    )
    def sc_aggregate(h_hbm, zeros_hbm, src_hbm, dstw_hbm, out_hbm,
                     src_v, dw, rows_v, agg_sh, isem, sem0, sem1, sem2):
        c = lax.axis_index("c")
        s = lax.axis_index("s")
        wid = s * _NC + c
        sems = (sem0, sem1, sem2)

        pltpu.sync_copy(src_hbm.at[wid], src_v)
        pltpu.sync_copy(dstw_hbm.at[wid, 0], dw.at[0])

        @pl.when(s == 0)
        def _():
            @pl.when(c == 0)
            def _():
                pltpu.sync_copy(h_hbm, agg_sh)

            @pl.when(c == 1)
            def _():
                pltpu.sync_copy(zeros_hbm, agg_sh)

        plsc.subcore_barrier()

        def gather(j, b):
            pltpu.async_copy(h_hbm.at[src_v.at[pl.ds(j * _CHUNK, _CHUNK)]],
                             rows_v.at[b], sems[b])

        def gwait(b):
            pltpu.make_async_copy(
                h_hbm.at[src_v.at[pl.ds(0, _CHUNK)]], rows_v.at[b],
                sems[b]).wait()

        def scatter(j, b):
            w = j // _W
            pltpu.sync_copy(rows_v.at[b],
                            agg_sh.at[dw.at[w % 2, j % _W]], add=True)

        def win_events(j):
            @pl.when(jnp.logical_and(j % _W == 0, j > 0))
            def _():
                pltpu.make_async_copy(dstw_hbm.at[wid, 0], dw.at[0],
                                      isem).wait()

            @pl.when(jnp.logical_and(j % _W == 0, j < (_NWIN - 1) * _W))
            def _():
                w1 = j // _W + 1
                pltpu.async_copy(dstw_hbm.at[wid, w1], dw.at[w1 % 2], isem)

        gather(0, 0)
        gather(1, 1)
        gather(2, 2)

        def body(k, carry):
            for t in range(3):
                j = 3 * k + t
                win_events(j)
                gwait(t)
                scatter(j, t)
                gather(j + 3, t)
            return carry

        lax.fori_loop(0, (_NCHUNK - 5) // 3, body, 0)

        jt = _NCHUNK - 5
        gwait(0)
        scatter(jt, 0)
        gather(jt + 3, 0)
        gwait(1)
        scatter(jt + 1, 1)
        gather(jt + 4, 1)
        gwait(2)
        scatter(jt + 2, 2)
        gwait(0)
        scatter(jt + 3, 0)
        gwait(1)
        scatter(jt + 4, 1)

        plsc.subcore_barrier()

        @pl.when(s == 0)
        def _():
            pltpu.sync_copy(agg_sh, out_hbm.at[c])

    return sc_aggregate


def _tc_mlp_body(p_ref, W1_ref, b1_ref, gamma_ref, beta_ref, W2_ref, b2_ref,
                 out_ref, t_sc, sum_sc, sq_sc, ss_sc):
    ph = pl.program_id(0)
    i = pl.program_id(1)

    @pl.when(ph == 0)
    def _():
        h = p_ref[0] + p_ref[1]
        t = jnp.dot(h, W1_ref[...], preferred_element_type=jnp.float32)
        t = t + b1_ref[...]
        t_sc[pl.ds(i * _BLK, _BLK), :] = t

        @pl.when(i == 0)
        def _():
            sum_sc[...] = jnp.zeros_like(sum_sc)
            sq_sc[...] = jnp.zeros_like(sq_sc)

        sum_sc[...] += jnp.sum(t, axis=0, keepdims=True)
        sq_sc[...] += jnp.sum(t * t, axis=0, keepdims=True)

        @pl.when(i == _NBLK - 1)
        def _():
            mean = sum_sc[...] / _N
            var = sq_sc[...] / _N - mean * mean
            scale = gamma_ref[...] * lax.rsqrt(var + 1e-5)
            ss_sc[0:1, :] = scale
            ss_sc[1:2, :] = beta_ref[...] - mean * scale

    @pl.when(ph == 1)
    def _():
        t = t_sc[pl.ds(i * _BLK, _BLK), :]
        u = jnp.maximum(t * ss_sc[0:1, :] + ss_sc[1:2, :], 0.0)
        y = jnp.dot(u, W2_ref[...], preferred_element_type=jnp.float32)
        out_ref[...] = jnp.maximum(y + b2_ref[...], 0.0)


_vec_spec = pl.BlockSpec((1, _D), lambda ph, i: (0, 0))
_mat_spec = pl.BlockSpec((_D, _D), lambda ph, i: (0, 0))
_p_spec = pl.BlockSpec((2, _BLK, _D), lambda ph, i: (0, i * (1 - ph), 0))

_tc_mlp = pl.pallas_call(
    _tc_mlp_body,
    grid=(2, _NBLK),
    in_specs=[_p_spec, _mat_spec, _vec_spec, _vec_spec, _vec_spec, _mat_spec,
              _vec_spec],
    out_specs=pl.BlockSpec((_BLK, _D), lambda ph, i: (i, 0)),
    out_shape=jax.ShapeDtypeStruct((_N, _D), jnp.float32),
    scratch_shapes=[
        pltpu.VMEM((_N, _D), jnp.float32),
        pltpu.VMEM((1, _D), jnp.float32),
        pltpu.VMEM((1, _D), jnp.float32),
        pltpu.VMEM((2, _D), jnp.float32),
    ],
    compiler_params=pltpu.CompilerParams(
        dimension_semantics=("arbitrary", "arbitrary")),
)


def _tc_final_body(p_ref, W1_ref, b1_ref, gamma_ref, beta_ref, W2_ref, b2_ref,
                   bm_ref, mW1_ref, mb1_ref, mW2_ref, mb2_ref,
                   out_ref, t_sc, sum_sc, sq_sc, ss_sc, pool_sc):
    ph = pl.program_id(0)
    i = pl.program_id(1)

    @pl.when(ph == 0)
    def _():
        h = p_ref[0] + p_ref[1]
        t = jnp.dot(h, W1_ref[...], preferred_element_type=jnp.float32)
        t = t + b1_ref[...]
        t_sc[pl.ds(i * _BLK, _BLK), :] = t

        @pl.when(i == 0)
        def _():
            sum_sc[...] = jnp.zeros_like(sum_sc)
            sq_sc[...] = jnp.zeros_like(sq_sc)

        sum_sc[...] += jnp.sum(t, axis=0, keepdims=True)
        sq_sc[...] += jnp.sum(t * t, axis=0, keepdims=True)

        @pl.when(i == _NBLK - 1)
        def _():
            mean = sum_sc[...] / _N
            var = sq_sc[...] / _N - mean * mean
            scale = gamma_ref[...] * lax.rsqrt(var + 1e-5)
            ss_sc[0:1, :] = scale
            ss_sc[1:2, :] = beta_ref[...] - mean * scale

    @pl.when(ph == 1)
    def _():
        t = t_sc[pl.ds(i * _BLK, _BLK), :]
        u = jnp.maximum(t * ss_sc[0:1, :] + ss_sc[1:2, :], 0.0)
        y = jnp.dot(u, W2_ref[...], preferred_element_type=jnp.float32)
        y = jnp.maximum(y + b2_ref[...], 0.0)

        oh = (bm_ref[0] == lax.broadcasted_iota(jnp.int32, (_G, _BLK), 0))
        contrib = jnp.dot(oh.astype(jnp.float32), y,
                          preferred_element_type=jnp.float32)

        @pl.when(i == 0)
        def _():
            pool_sc[...] = contrib

        @pl.when(i > 0)
        def _():
            pool_sc[...] += contrib

        @pl.when(i == _NBLK - 1)
        def _():
            h2 = jnp.dot(pool_sc[...], mW1_ref[...],
                         preferred_element_type=jnp.float32)
            h2 = jnp.maximum(h2 + mb1_ref[...], 0.0)
            o = jnp.dot(h2, mW2_ref[...], preferred_element_type=jnp.float32)
            out_ref[...] = o + mb2_ref[...]


_tc_final = pl.pallas_call(
    _tc_final_body,
    grid=(2, _NBLK),
    in_specs=[_p_spec, _mat_spec, _vec_spec, _vec_spec, _vec_spec, _mat_spec,
              _vec_spec,
              pl.BlockSpec((1, 1, _BLK), lambda ph, i: (i, 0, 0)),
              _mat_spec, _vec_spec, _mat_spec, _vec_spec],
    out_specs=pl.BlockSpec((_G, _D), lambda ph, i: (0, 0)),
    out_shape=jax.ShapeDtypeStruct((_G, _D), jnp.float32),
    scratch_shapes=[
        pltpu.VMEM((_N, _D), jnp.float32),
        pltpu.VMEM((1, _D), jnp.float32),
        pltpu.VMEM((1, _D), jnp.float32),
        pltpu.VMEM((2, _D), jnp.float32),
        pltpu.VMEM((_G, _D), jnp.float32),
    ],
    compiler_params=pltpu.CompilerParams(
        dimension_semantics=("arbitrary", "arbitrary")),
)


def kernel(x, edge_index, batch,
           conv0_W1, conv0_b1, conv0_gamma, conv0_beta, conv0_W2, conv0_b2,
           conv1_W1, conv1_b1, conv1_gamma, conv1_beta, conv1_W2, conv1_b2,
           conv2_W1, conv2_b1, conv2_gamma, conv2_beta, conv2_W2, conv2_b2,
           mlp_W1, mlp_b1, mlp_W2, mlp_b2):
    src = edge_index[0].reshape(_NW, _NCHUNK, _CHUNK)
    dst = edge_index[1].reshape(_NW, _NCHUNK, _CHUNK)
    srcall = edge_index[0].reshape(_NW, _EPT)
    dstw = dst.reshape(_NW, _NWIN, _W, _CHUNK)
    zeros = jnp.zeros((_N, _D), jnp.float32)
    bm = batch.reshape(_NBLK, 1, _BLK)
    r1 = lambda v: v.reshape(1, _D)

    sc_aggregate = _get_sc_aggregate()
    h = x
    for (W1, b1, g, be, W2, b2) in (
        (conv0_W1, conv0_b1, conv0_gamma, conv0_beta, conv0_W2, conv0_b2),
        (conv1_W1, conv1_b1, conv1_gamma, conv1_beta, conv1_W2, conv1_b2),
    ):
        parts = sc_aggregate(h, zeros, srcall, dstw)
        h = _tc_mlp(parts, W1, r1(b1), r1(g), r1(be), W2, r1(b2))

    parts = sc_aggregate(h, zeros, srcall, dstw)
    out = _tc_final(parts, conv2_W1, r1(conv2_b1), r1(conv2_gamma),
                    r1(conv2_beta), conv2_W2, r1(conv2_b2),
                    bm, mlp_W1, r1(mlp_b1), mlp_W2, r1(mlp_b2))
    return out


# speedup vs baseline: 12.4391x; 1.0209x over previous
import functools

import jax
import jax.numpy as jnp
from jax import lax
from jax.experimental import pallas as pl
from jax.experimental.pallas import tpu as pltpu
from jax.experimental.pallas import tpu_sc as plsc

_N = 10000
_E = 320000
_D = 128
_G = 64

_NC = 2
_NS = 16
_NW = _NC * _NS
_EPT = _E // _NW
_CHUNK = 80
_NCHUNK = _EPT // _CHUNK
_W = 5
_NWIN = _NCHUNK // _W

_BLK = 1000
_NBLK = _N // _BLK
_RPT = 624
_RREM = _N - _NS * _RPT


@functools.cache
def _get_sc_aggregate():
    mesh = plsc.VectorSubcoreMesh(
        core_axis_name="c", subcore_axis_name="s",
        num_cores=_NC, num_subcores=_NS)

    @functools.partial(
        pl.kernel,
        out_type=jax.ShapeDtypeStruct((2, _N, _D), jnp.float32),
        mesh=mesh,
        scratch_types=[
            pltpu.VMEM((3, _W, _CHUNK), jnp.int32),
            pltpu.VMEM((3, _W, _CHUNK), jnp.int32),
            pltpu.VMEM((4, _CHUNK, _D), jnp.float32),
            pltpu.VMEM_SHARED((_N, _D), jnp.float32),
            pltpu.SemaphoreType.DMA,
            pltpu.SemaphoreType.DMA,
            pltpu.SemaphoreType.DMA,
            pltpu.SemaphoreType.DMA,
            pltpu.SemaphoreType.DMA,
            pltpu.SemaphoreType.DMA,
        ],
    )
    def sc_aggregate(h_hbm, zeros_hbm, srcw_hbm, dstw_hbm, out_hbm,
                     sw, dw, rows_v, agg_sh, isem, jsem,
                     sem0, sem1, sem2, sem3):
        c = lax.axis_index("c")
        s = lax.axis_index("s")
        wid = s * _NC + c
        sems = (sem0, sem1, sem2, sem3)

        pltpu.sync_copy(srcw_hbm.at[wid, 0], sw.at[0])
        pltpu.sync_copy(dstw_hbm.at[wid, 0], dw.at[0])
        pltpu.sync_copy(srcw_hbm.at[wid, 1], sw.at[1])
        pltpu.sync_copy(dstw_hbm.at[wid, 1], dw.at[1])
        pltpu.async_copy(srcw_hbm.at[wid, 2], sw.at[2], jsem)
        pltpu.async_copy(dstw_hbm.at[wid, 2], dw.at[2], isem)

        @pl.when(s == 0)
        def _():
            @pl.when(c == 0)
            def _():
                pltpu.sync_copy(h_hbm, agg_sh)

            @pl.when(c == 1)
            def _():
                pltpu.sync_copy(zeros_hbm, agg_sh)

        plsc.subcore_barrier()

        def gather(j, b):
            w = j // _W
            pltpu.async_copy(h_hbm.at[sw.at[w % 3, j % _W]],
                             rows_v.at[b], sems[b])

        def gwait(b):
            pltpu.make_async_copy(
                h_hbm.at[sw.at[0, 0]], rows_v.at[b], sems[b]).wait()

        def scatter(j, b):
            w = j // _W
            pltpu.sync_copy(rows_v.at[b],
                            agg_sh.at[dw.at[w % 3, j % _W]], add=True)

        def win_events(j):
            @pl.when(jnp.logical_and(j % _W == 0,
                                     jnp.logical_and(j > 0,
                                                     j <= (_NWIN - 2) * _W)))
            def _():
                pltpu.make_async_copy(srcw_hbm.at[wid, 0], sw.at[0],
                                      jsem).wait()
                pltpu.make_async_copy(dstw_hbm.at[wid, 0], dw.at[0],
                                      isem).wait()

            @pl.when(jnp.logical_and(j % _W == 0,
                                     jnp.logical_and(j > 0,
                                                     j <= (_NWIN - 3) * _W)))
            def _():
                w2 = j // _W + 2
                pltpu.async_copy(srcw_hbm.at[wid, w2], sw.at[w2 % 3], jsem)
                pltpu.async_copy(dstw_hbm.at[wid, w2], dw.at[w2 % 3], isem)

        gather(0, 0)
        gather(1, 1)
        gather(2, 2)
        gather(3, 3)

        def body(k, carry):
            for t in range(4):
                j = 4 * k + t
                win_events(j)
                gwait(t)
                scatter(j, t)
                gather(j + 4, t)
            return carry

        lax.fori_loop(0, (_NCHUNK - 5) // 4, body, 0)

        jt = _NCHUNK - 5
        gwait(0)
        scatter(jt, 0)
        gather(jt + 4, 0)
        gwait(1)
        scatter(jt + 1, 1)
        gwait(2)
        scatter(jt + 2, 2)
        gwait(3)
        scatter(jt + 3, 3)
        gwait(0)
        scatter(jt + 4, 0)

        plsc.subcore_barrier()

        @pl.when(s == 0)
        def _():
            pltpu.sync_copy(agg_sh, out_hbm.at[c])

    return sc_aggregate


def _tc_mlp_body(p_ref, W1_ref, b1_ref, gamma_ref, beta_ref, W2_ref, b2_ref,
                 out_ref, t_sc, sum_sc, sq_sc, ss_sc):
    ph = pl.program_id(0)
    i = pl.program_id(1)

    @pl.when(ph == 0)
    def _():
        h = p_ref[0] + p_ref[1]
        t = jnp.dot(h, W1_ref[...], preferred_element_type=jnp.float32)
        t = t + b1_ref[...]
        t_sc[pl.ds(i * _BLK, _BLK), :] = t

        @pl.when(i == 0)
        def _():
            sum_sc[...] = jnp.zeros_like(sum_sc)
            sq_sc[...] = jnp.zeros_like(sq_sc)

        sum_sc[...] += jnp.sum(t, axis=0, keepdims=True)
        sq_sc[...] += jnp.sum(t * t, axis=0, keepdims=True)

        @pl.when(i == _NBLK - 1)
        def _():
            mean = sum_sc[...] / _N
            var = sq_sc[...] / _N - mean * mean
            scale = gamma_ref[...] * lax.rsqrt(var + 1e-5)
            ss_sc[0:1, :] = scale
            ss_sc[1:2, :] = beta_ref[...] - mean * scale

    @pl.when(ph == 1)
    def _():
        t = t_sc[pl.ds(i * _BLK, _BLK), :]
        u = jnp.maximum(t * ss_sc[0:1, :] + ss_sc[1:2, :], 0.0)
        y = jnp.dot(u, W2_ref[...], preferred_element_type=jnp.float32)
        out_ref[...] = jnp.maximum(y + b2_ref[...], 0.0)


_vec_spec = pl.BlockSpec((1, _D), lambda ph, i: (0, 0))
_mat_spec = pl.BlockSpec((_D, _D), lambda ph, i: (0, 0))
_p_spec = pl.BlockSpec((2, _BLK, _D), lambda ph, i: (0, i * (1 - ph), 0))

_tc_mlp = pl.pallas_call(
    _tc_mlp_body,
    grid=(2, _NBLK),
    in_specs=[_p_spec, _mat_spec, _vec_spec, _vec_spec, _vec_spec, _mat_spec,
              _vec_spec],
    out_specs=pl.BlockSpec((_BLK, _D), lambda ph, i: (i, 0)),
    out_shape=jax.ShapeDtypeStruct((_N, _D), jnp.float32),
    scratch_shapes=[
        pltpu.VMEM((_N, _D), jnp.float32),
        pltpu.VMEM((1, _D), jnp.float32),
        pltpu.VMEM((1, _D), jnp.float32),
        pltpu.VMEM((2, _D), jnp.float32),
    ],
    compiler_params=pltpu.CompilerParams(
        dimension_semantics=("arbitrary", "arbitrary")),
)


def _tc_final_body(p_ref, W1_ref, b1_ref, gamma_ref, beta_ref, W2_ref, b2_ref,
                   bm_ref, mW1_ref, mb1_ref, mW2_ref, mb2_ref,
                   out_ref, t_sc, sum_sc, sq_sc, ss_sc, pool_sc):
    ph = pl.program_id(0)
    i = pl.program_id(1)

    @pl.when(ph == 0)
    def _():
        h = p_ref[0] + p_ref[1]
        t = jnp.dot(h, W1_ref[...], preferred_element_type=jnp.float32)
        t = t + b1_ref[...]
        t_sc[pl.ds(i * _BLK, _BLK), :] = t

        @pl.when(i == 0)
        def _():
            sum_sc[...] = jnp.zeros_like(sum_sc)
            sq_sc[...] = jnp.zeros_like(sq_sc)

        sum_sc[...] += jnp.sum(t, axis=0, keepdims=True)
        sq_sc[...] += jnp.sum(t * t, axis=0, keepdims=True)

        @pl.when(i == _NBLK - 1)
        def _():
            mean = sum_sc[...] / _N
            var = sq_sc[...] / _N - mean * mean
            scale = gamma_ref[...] * lax.rsqrt(var + 1e-5)
            ss_sc[0:1, :] = scale
            ss_sc[1:2, :] = beta_ref[...] - mean * scale

    @pl.when(ph == 1)
    def _():
        t = t_sc[pl.ds(i * _BLK, _BLK), :]
        u = jnp.maximum(t * ss_sc[0:1, :] + ss_sc[1:2, :], 0.0)
        y = jnp.dot(u, W2_ref[...], preferred_element_type=jnp.float32)
        y = jnp.maximum(y + b2_ref[...], 0.0)

        oh = (bm_ref[0] == lax.broadcasted_iota(jnp.int32, (_G, _BLK), 0))
        contrib = jnp.dot(oh.astype(jnp.float32), y,
                          preferred_element_type=jnp.float32)

        @pl.when(i == 0)
        def _():
            pool_sc[...] = contrib

        @pl.when(i > 0)
        def _():
            pool_sc[...] += contrib

        @pl.when(i == _NBLK - 1)
        def _():
            h2 = jnp.dot(pool_sc[...], mW1_ref[...],
                         preferred_element_type=jnp.float32)
            h2 = jnp.maximum(h2 + mb1_ref[...], 0.0)
            o = jnp.dot(h2, mW2_ref[...], preferred_element_type=jnp.float32)
            out_ref[...] = o + mb2_ref[...]


_tc_final = pl.pallas_call(
    _tc_final_body,
    grid=(2, _NBLK),
    in_specs=[_p_spec, _mat_spec, _vec_spec, _vec_spec, _vec_spec, _mat_spec,
              _vec_spec,
              pl.BlockSpec((1, 1, _BLK), lambda ph, i: (i, 0, 0)),
              _mat_spec, _vec_spec, _mat_spec, _vec_spec],
    out_specs=pl.BlockSpec((_G, _D), lambda ph, i: (0, 0)),
    out_shape=jax.ShapeDtypeStruct((_G, _D), jnp.float32),
    scratch_shapes=[
        pltpu.VMEM((_N, _D), jnp.float32),
        pltpu.VMEM((1, _D), jnp.float32),
        pltpu.VMEM((1, _D), jnp.float32),
        pltpu.VMEM((2, _D), jnp.float32),
        pltpu.VMEM((_G, _D), jnp.float32),
    ],
    compiler_params=pltpu.CompilerParams(
        dimension_semantics=("arbitrary", "arbitrary")),
)


def kernel(x, edge_index, batch,
           conv0_W1, conv0_b1, conv0_gamma, conv0_beta, conv0_W2, conv0_b2,
           conv1_W1, conv1_b1, conv1_gamma, conv1_beta, conv1_W2, conv1_b2,
           conv2_W1, conv2_b1, conv2_gamma, conv2_beta, conv2_W2, conv2_b2,
           mlp_W1, mlp_b1, mlp_W2, mlp_b2):
    srcw = edge_index[0].reshape(_NW, _NWIN, _W, _CHUNK)
    dstw = edge_index[1].reshape(_NW, _NWIN, _W, _CHUNK)
    zeros = jnp.zeros((_N, _D), jnp.float32)
    bm = batch.reshape(_NBLK, 1, _BLK)
    r1 = lambda v: v.reshape(1, _D)

    sc_aggregate = _get_sc_aggregate()
    h = x
    for (W1, b1, g, be, W2, b2) in (
        (conv0_W1, conv0_b1, conv0_gamma, conv0_beta, conv0_W2, conv0_b2),
        (conv1_W1, conv1_b1, conv1_gamma, conv1_beta, conv1_W2, conv1_b2),
    ):
        parts = sc_aggregate(h, zeros, srcw, dstw)
        h = _tc_mlp(parts, W1, r1(b1), r1(g), r1(be), W2, r1(b2))

    parts = sc_aggregate(h, zeros, srcw, dstw)
    out = _tc_final(parts, conv2_W1, r1(conv2_b1), r1(conv2_gamma),
                    r1(conv2_beta), conv2_W2, r1(conv2_b2),
                    bm, mlp_W1, r1(mlp_b1), mlp_W2, r1(mlp_b2))
    return out
